# baseline (speedup 1.0000x reference)
import functools

import jax
import jax.numpy as jnp
from jax import lax
from jax.experimental import pallas as pl
from jax.experimental.pallas import tpu as pltpu
from jax.experimental.pallas import tpu_sc as plsc

_R = 128
_ROWS = 51200
_CHUNK = 128
_NUM_CHUNKS = _ROWS // _CHUNK
_NEG = -100000.0
_L = 16


def _bcast_lane(vec, lane):
    idx = jnp.full((_L, 1), lane, jnp.int32)
    dnums = lax.GatherDimensionNumbers(
        offset_dims=(), collapsed_slice_dims=(0,), start_index_map=(0,))
    return lax.gather(vec, idx, dnums, (1,),
                      mode=lax.GatherScatterMode.PROMISE_IN_BOUNDS)


def _build_sc_call():
    info = plsc.get_sparse_core_info()
    nc, ns = info.num_cores, info.num_subcores
    nw = nc * ns
    base_chunks, extra = divmod(_NUM_CHUNKS, nw)
    mesh = plsc.VectorSubcoreMesh(core_axis_name="c", subcore_axis_name="s")

    @functools.partial(
        pl.kernel,
        mesh=mesh,
        out_type=jax.ShapeDtypeStruct((_ROWS, _R), jnp.float32),
        scratch_types=[
            pltpu.VMEM((_CHUNK,), jnp.int32),
            pltpu.VMEM((_CHUNK,), jnp.int32),
            pltpu.VMEM((_CHUNK, _R), jnp.float32),
            pltpu.VMEM((_CHUNK, _R), jnp.float32),
            pltpu.VMEM((_R,), jnp.float32),
            pltpu.SemaphoreType.DMA,
        ],
    )
    def sc_kernel(idx_hbm, mask_hbm, priors_hbm, uncond_hbm, table_hbm,
                  out_hbm, idx_v, mask_v, rows_v, pri_v, unc_v, sem):
        wid = lax.axis_index("s") * nc + lax.axis_index("c")
        n_w = jnp.where(wid < extra, base_chunks + 1, base_chunks)
        start_w = wid * base_chunks + jnp.minimum(wid, extra)
        pltpu.sync_copy(uncond_hbm, unc_v)

        def chunk_body(j, carry):
            base = (start_w + j) * _CHUNK
            pltpu.sync_copy(idx_hbm.at[pl.ds(base, _CHUNK)], idx_v)
            pltpu.sync_copy(mask_hbm.at[pl.ds(base, _CHUNK)], mask_v)
            pltpu.sync_copy(priors_hbm.at[pl.ds(base, _CHUNK)], pri_v)
            pltpu.async_copy(table_hbm.at[idx_v], rows_v, sem).wait()

            for mg in range(_CHUNK // _L):
                mv = mask_v[pl.ds(mg * _L, _L)]

                def row_body(r2, carry2, mv=mv, mg=mg):
                    r = mg * _L + r2
                    mf = jnp.minimum(_bcast_lane(mv, r2), 1).astype(jnp.float32)
                    neg = (mf - 1.0) * (-_NEG)
                    for g in range(_R // _L):
                        sl = pl.ds(g * _L, _L)
                        rows_v[r, sl] = ((rows_v[r, sl] + unc_v[sl]) * mf
                                         + (pri_v[r, sl] + neg))
                    return carry2

                lax.fori_loop(0, _L, row_body, 0)
            pltpu.sync_copy(rows_v, out_hbm.at[pl.ds(base, _CHUNK)])
            return carry

        lax.fori_loop(0, n_w, chunk_body, 0)

    return sc_kernel


_SC_CALL = _build_sc_call()


def kernel(cond_inds, node_mask, full_logit_priors, unconditionals, conditionals):
    b = cond_inds.shape[0]
    idx = cond_inds.reshape(-1)
    mask = node_mask.reshape(-1)
    priors = full_logit_priors.reshape(-1, _R)
    out = _SC_CALL(idx, mask, priors, unconditionals, conditionals)
    masked_policy_logits = out.reshape(b, -1)
    used_priors = full_logit_priors.reshape(b, -1)
    return (masked_policy_logits, used_priors)


# speedup vs baseline: 1.3388x; 1.3388x over previous
import functools

import jax
import jax.numpy as jnp
from jax import lax
from jax.experimental import pallas as pl
from jax.experimental.pallas import tpu as pltpu
from jax.experimental.pallas import tpu_sc as plsc

_R = 128
_ROWS = 51200
_CHUNK = 80
_L = 16
_BIG = 100000.0


def _bcast_lane(vec, lane):
    idx = jnp.full((_L, 1), lane, jnp.int32)
    dnums = lax.GatherDimensionNumbers(
        offset_dims=(), collapsed_slice_dims=(0,), start_index_map=(0,))
    return lax.gather(vec, idx, dnums, (1,),
                      mode=lax.GatherScatterMode.PROMISE_IN_BOUNDS)


def _build_sc_call():
    info = plsc.get_sparse_core_info()
    nc, ns = info.num_cores, info.num_subcores
    nw = nc * ns
    rows_w = _ROWS // nw
    n_chunks = rows_w // _CHUNK
    mesh = plsc.VectorSubcoreMesh(core_axis_name="c", subcore_axis_name="s")

    @functools.partial(
        pl.kernel,
        mesh=mesh,
        out_type=jax.ShapeDtypeStruct((_ROWS, _R), jnp.float32),
        scratch_types=[
            pltpu.VMEM((3, _CHUNK), jnp.int32),
            pltpu.VMEM((3, _CHUNK), jnp.int32),
            pltpu.VMEM((3, _CHUNK, _R), jnp.float32),
            pltpu.VMEM((2, _CHUNK, _R), jnp.float32),
            pltpu.VMEM((_R,), jnp.float32),
            pltpu.SemaphoreType.DMA((3,)),
            pltpu.SemaphoreType.DMA((2,)),
            pltpu.SemaphoreType.DMA((2,)),
        ],
    )
    def sc_kernel(idx_hbm, mask_hbm, priors_hbm, uncond_hbm, table_hbm,
                  out_hbm, idx_v, mask_v, pri_v, rows_v, unc_v,
                  sem_lin, sem_g, sem_out):
        wid = lax.axis_index("s") * nc + lax.axis_index("c")
        row0 = wid * rows_w
        pltpu.sync_copy(uncond_hbm, unc_v)
        unc = [unc_v[pl.ds(g * _L, _L)] for g in range(_R // _L)]

        def issue_lin(j):
            b3 = j % 3
            base = row0 + j * _CHUNK
            return (
                pltpu.async_copy(idx_hbm.at[pl.ds(base, _CHUNK)],
                                 idx_v.at[b3], sem_lin.at[b3]),
                pltpu.async_copy(mask_hbm.at[pl.ds(base, _CHUNK)],
                                 mask_v.at[b3], sem_lin.at[b3]),
                pltpu.async_copy(priors_hbm.at[pl.ds(base, _CHUNK)],
                                 pri_v.at[b3], sem_lin.at[b3]),
            )

        def issue_gather(j):
            b3, b2 = j % 3, j % 2
            return pltpu.async_copy(table_hbm.at[idx_v.at[b3]],
                                    rows_v.at[b2], sem_g.at[b2])

        def issue_out(j):
            b2 = j % 2
            base = row0 + j * _CHUNK
            return pltpu.async_copy(rows_v.at[b2],
                                    out_hbm.at[pl.ds(base, _CHUNK)],
                                    sem_out.at[b2])

        def compute(j):
            b3, b2 = j % 3, j % 2

            def row_body(r, carry):
                mv = mask_v[b3, pl.ds((r // _L) * _L, _L)]
                mf = jnp.minimum(_bcast_lane(mv, r % _L), 1).astype(jnp.float32)
                neg = (mf - 1.0) * _BIG
                for g in range(_R // _L):
                    sl = pl.ds(g * _L, _L)
                    rows_v[b2, r, sl] = ((rows_v[b2, r, sl] + unc[g]) * mf
                                         + (pri_v[b3, r, sl] + neg))
                return carry

            lax.fori_loop(0, _CHUNK, row_body, 0)

        handles = {}
        handles[("lin", 0)] = issue_lin(0)
        handles[("lin", 1)] = issue_lin(1)
        for h in handles.pop(("lin", 0)):
            h.wait()
        handles[("g", 0)] = issue_gather(0)

        for j in range(n_chunks):
            if j + 2 < n_chunks:
                handles[("lin", j + 2)] = issue_lin(j + 2)
            if j + 1 < n_chunks:
                for h in handles.pop(("lin", j + 1)):
                    h.wait()
                if j >= 1:
                    handles.pop(("out", j - 1)).wait()
                handles[("g", j + 1)] = issue_gather(j + 1)
            handles.pop(("g", j)).wait()
            compute(j)
            handles[("out", j)] = issue_out(j)

        handles.pop(("out", n_chunks - 2)).wait()
        handles.pop(("out", n_chunks - 1)).wait()

    return sc_kernel


_SC_CALL = _build_sc_call()


def kernel(cond_inds, node_mask, full_logit_priors, unconditionals, conditionals):
    b = cond_inds.shape[0]
    idx = cond_inds.reshape(-1)
    mask = node_mask.reshape(-1)
    priors = full_logit_priors.reshape(-1, _R)
    out = _SC_CALL(idx, mask, priors, unconditionals, conditionals)
    masked_policy_logits = out.reshape(b, -1)
    used_priors = full_logit_priors.reshape(b, -1)
    return (masked_policy_logits, used_priors)


# speedup vs baseline: 1.8643x; 1.3926x over previous
import functools

import jax
import jax.numpy as jnp
from jax import lax
from jax.experimental import pallas as pl
from jax.experimental.pallas import tpu as pltpu
from jax.experimental.pallas import tpu_sc as plsc

_R = 128
_B = 1024
_N = 50
_L = 16
_BB = 2
_PAD = 104
_BIG = 100000.0


def _bcast_lane(vec, lane):
    idx = jnp.full((_L, 1), lane, jnp.int32)
    dnums = lax.GatherDimensionNumbers(
        offset_dims=(), collapsed_slice_dims=(0,), start_index_map=(0,))
    return lax.gather(vec, idx, dnums, (1,),
                      mode=lax.GatherScatterMode.PROMISE_IN_BOUNDS)


def _build_sc_call():
    info = plsc.get_sparse_core_info()
    nc, ns = info.num_cores, info.num_subcores
    nw = nc * ns
    bat_w = _B // nw
    n_chunks = bat_w // _BB
    mesh = plsc.VectorSubcoreMesh(core_axis_name="c", subcore_axis_name="s")

    @functools.partial(
        pl.kernel,
        mesh=mesh,
        out_type=jax.ShapeDtypeStruct((_B, _N * _R), jnp.float32),
        scratch_types=[
            pltpu.VMEM((3, _BB, _R), jnp.float32),
            pltpu.VMEM((3, _BB, _R), jnp.float32),
            pltpu.VMEM((2, _BB, 64), jnp.int32),
            pltpu.VMEM((3, _PAD, _R), jnp.float32),
            pltpu.VMEM((2, _PAD, _R), jnp.float32),
            pltpu.VMEM((_R,), jnp.float32),
            pltpu.SemaphoreType.DMA((3,)),
            pltpu.SemaphoreType.DMA((2,)),
            pltpu.SemaphoreType.DMA((2,)),
        ],
        compiler_params=pltpu.CompilerParams(use_tc_tiling_on_sc=True),
    )
    def sc_kernel(cond_hbm, nmask_hbm, priors_hbm, uncond_hbm, table_hbm,
                  out_hbm, idxf_v, mask_v, idxi_v, pri_v, rows_v, unc_v,
                  sem_lin, sem_g, sem_out):
        wid = lax.axis_index("s") * nc + lax.axis_index("c")
        bat0 = wid * bat_w
        pltpu.sync_copy(uncond_hbm, unc_v)
        unc = [unc_v[pl.ds(g * _L, _L)] for g in range(_R // _L)]

        def issue_lin(j):
            b3 = j % 3
            b = bat0 + j * _BB
            return (
                pltpu.async_copy(cond_hbm.at[pl.ds(b, _BB)],
                                 idxf_v.at[b3], sem_lin.at[b3]),
                pltpu.async_copy(nmask_hbm.at[pl.ds(b, _BB)],
                                 mask_v.at[b3], sem_lin.at[b3]),
                pltpu.async_copy(
                    priors_hbm.at[pl.ds(b, _BB)],
                    pri_v.at[b3, pl.ds(0, _BB * _N)].reshape(_BB, _N, _R),
                    sem_lin.at[b3]),
            )

        def convert_idx(j):
            b3, b2 = j % 3, j % 2
            for i in range(_BB):
                for t in range(4):
                    sl = pl.ds(t * _L, _L)
                    idxi_v[b2, i, sl] = idxf_v[b3, i, sl].astype(jnp.int32)

        def issue_gather(j):
            b2 = j % 2
            return tuple(
                pltpu.async_copy(table_hbm.at[idxi_v.at[b2, i, pl.ds(0, _N)]],
                                 rows_v.at[b2, pl.ds(i * _N, _N)],
                                 sem_g.at[b2])
                for i in range(_BB))

        def issue_out(j):
            b2 = j % 2
            return pltpu.async_copy(
                rows_v.at[b2, pl.ds(0, _BB * _N)].reshape(_BB, _N * _R),
                out_hbm.at[pl.ds(bat0 + j * _BB, _BB)],
                sem_out.at[b2])

        def compute(j):
            b3, b2 = j % 3, j % 2

            def row_body(r, carry):
                i = r // _N
                n = r - i * _N
                mv = mask_v[b3, i, pl.ds((n // _L) * _L, _L)]
                mf = jnp.minimum(_bcast_lane(mv, n % _L), 1.0)
                neg = (mf - 1.0) * _BIG
                for g in range(_R // _L):
                    sl = pl.ds(g * _L, _L)
                    rows_v[b2, r, sl] = ((rows_v[b2, r, sl] + unc[g]) * mf
                                         + (pri_v[b3, r, sl] + neg))
                return carry

            lax.fori_loop(0, _BB * _N, row_body, 0)

        handles = {}
        handles[("lin", 0)] = issue_lin(0)
        handles[("lin", 1)] = issue_lin(1)
        for h in handles.pop(("lin", 0)):
            h.wait()
        convert_idx(0)
        handles[("g", 0)] = issue_gather(0)

        for j in range(n_chunks):
            if j + 2 < n_chunks:
                handles[("lin", j + 2)] = issue_lin(j + 2)
            if j + 1 < n_chunks:
                for h in handles.pop(("lin", j + 1)):
                    h.wait()
                convert_idx(j + 1)
                if j >= 1:
                    handles.pop(("out", j - 1)).wait()
                handles[("g", j + 1)] = issue_gather(j + 1)
            for h in handles.pop(("g", j)):
                h.wait()
            compute(j)
            handles[("out", j)] = issue_out(j)

        for key in sorted(handles, key=str):
            h = handles[key]
            for hh in (h if isinstance(h, tuple) else (h,)):
                hh.wait()

    return sc_kernel


def _used_body(pri_ref, out_ref):
    for n in range(_N):
        out_ref[:, pl.ds(n * _R, _R)] = pri_ref[:, n, :]


def _used_priors_tc(full_logit_priors):
    grid = (_B // 8,)
    return pl.pallas_call(
        _used_body,
        grid=grid,
        in_specs=[pl.BlockSpec((8, _N, _R), lambda b: (b, 0, 0))],
        out_specs=pl.BlockSpec((8, _N * _R), lambda b: (b, 0)),
        out_shape=jax.ShapeDtypeStruct((_B, _N * _R), jnp.float32),
    )(full_logit_priors)


_SC_CALL = _build_sc_call()


def kernel(cond_inds, node_mask, full_logit_priors, unconditionals, conditionals):
    pad = ((0, 0), (0, _R - _N))
    cond_f = jnp.pad(cond_inds.astype(jnp.float32), pad)
    mask_f = jnp.pad(node_mask.astype(jnp.float32), pad)
    masked_policy_logits = _SC_CALL(
        cond_f, mask_f, full_logit_priors, unconditionals, conditionals)
    used_priors = _used_priors_tc(full_logit_priors)
    return (masked_policy_logits, used_priors)
